# baseline (speedup 1.0000x reference)
import jax
import jax.numpy as jnp
from jax.experimental import pallas as pl
from jax.experimental.pallas import tpu as pltpu

_B = 32
_P = 8732
_C = 21
_O = 16
_P_PAD = 8960
_C_PAD = 24
_NEG_POS = 3
_THRESHOLD = 0.5
_V0 = 0.1
_V1 = 0.2


def _body(truths_ref, truthsT_ref, priorsT_ref, locT_ref, confT_ref,
          ll_ref, lc_ref, n_ref, mine_ref, kv_ref):
    b = pl.program_id(0)

    t = truths_ref[0]
    tx1 = t[:, 0:1]
    ty1 = t[:, 1:2]
    tx2 = t[:, 2:3]
    ty2 = t[:, 3:4]

    pr = priorsT_ref[...]
    pcx = pr[0:1]
    pcy = pr[1:2]
    pw = pr[2:3]
    ph = pr[3:4]
    px1 = pcx - pw * 0.5
    py1 = pcy - ph * 0.5
    px2 = pcx + pw * 0.5
    py2 = pcy + ph * 0.5

    iw = jnp.maximum(jnp.minimum(tx2, px2) - jnp.maximum(tx1, px1), 0.0)
    ih = jnp.maximum(jnp.minimum(ty2, py2) - jnp.maximum(ty1, py1), 0.0)
    inter = iw * ih
    area_t = (tx2 - tx1) * (ty2 - ty1)
    area_p = (px2 - px1) * (py2 - py1)
    ov = inter / (area_t + area_p - inter)

    o_iota = jax.lax.broadcasted_iota(jnp.int32, (_O, _P_PAD), 0)
    p_iota = jax.lax.broadcasted_iota(jnp.int32, (_O, _P_PAD), 1)

    bto = jnp.max(ov, axis=0, keepdims=True)
    bti = jnp.min(jnp.where(ov == bto, o_iota, _O), axis=0, keepdims=True)

    bpo = jnp.max(ov, axis=1, keepdims=True)
    bpi = jnp.min(jnp.where(ov == bpo, p_iota, _P_PAD), axis=1, keepdims=True)

    hit = p_iota == bpi
    any_hit = jnp.max(hit.astype(jnp.float32), axis=0, keepdims=True) > 0.0
    over_idx = jnp.max(jnp.where(hit, o_iota, -1), axis=0, keepdims=True)
    bto = jnp.where(any_hit, 2.0, bto)
    bti = jnp.where(any_hit, over_idx, bti)

    sel = (o_iota == bti).astype(jnp.float32)
    tT = truthsT_ref[0]
    mm = jax.lax.dot_general(tT, sel, (((1,), (0,)), ((), ())),
                             precision=jax.lax.Precision.HIGHEST,
                             preferred_element_type=jnp.float32)
    mx1 = mm[0:1]
    my1 = mm[1:2]
    mx2 = mm[2:3]
    my2 = mm[3:4]
    mlab = mm[4:5]

    conf_t = jnp.where(bto < _THRESHOLD, 0.0, mlab + 1.0)
    pos = conf_t > 0.0

    g_cx = ((mx1 + mx2) * 0.5 - pcx) / (_V0 * pw)
    g_cy = ((my1 + my2) * 0.5 - pcy) / (_V0 * ph)
    g_w = jnp.log((mx2 - mx1) / pw) / _V1
    g_h = jnp.log((my2 - my1) / ph) / _V1

    ld = locT_ref[0]

    def _sl1(d):
        a = jnp.abs(d)
        return jnp.where(a < 1.0, 0.5 * d * d, a - 0.5)

    sl1 = (_sl1(ld[0:1] - g_cx) + _sl1(ld[1:2] - g_cy)
           + _sl1(ld[2:3] - g_w) + _sl1(ld[3:4] - g_h))
    loss_l_row = jnp.sum(jnp.where(pos, sl1, 0.0))

    cf = confT_ref[0].T
    mrow = jnp.max(cf, axis=0, keepdims=True)
    s = jnp.sum(jnp.exp(cf - mrow), axis=0, keepdims=True)
    lse = jnp.log(s) + mrow
    c_iota = jax.lax.broadcasted_iota(jnp.int32, (_C_PAD, _P_PAD), 0)
    conf_t_i = conf_t.astype(jnp.int32)
    gathered = jnp.sum(jnp.where(c_iota == conf_t_i, cf, 0.0),
                       axis=0, keepdims=True)
    ce = lse - gathered

    valid = jax.lax.broadcasted_iota(jnp.int32, (1, _P_PAD), 1) < _P
    mining = jnp.where(pos | (~valid), 0.0, ce)
    ce_pos = jnp.sum(jnp.where(pos, ce, 0.0))
    npos = jnp.sum(pos.astype(jnp.float32))
    k = jnp.minimum(_NEG_POS * npos, float(_P - 1))

    mine_ref[pl.ds(b, 1), :] = mining
    kv_ref[pl.ds(b, 1), :] = jnp.full((1, 128), k, jnp.float32)

    @pl.when(b == 0)
    def _init():
        ll_ref[...] = jnp.zeros((1, 1), jnp.float32)
        lc_ref[...] = jnp.zeros((1, 1), jnp.float32)
        n_ref[...] = jnp.zeros((1, 1), jnp.float32)

    ll_ref[...] += jnp.full((1, 1), loss_l_row, jnp.float32)
    lc_ref[...] += jnp.full((1, 1), ce_pos, jnp.float32)
    n_ref[...] += jnp.full((1, 1), npos, jnp.float32)

    @pl.when(b == _B - 1)
    def _mine():
        mall = mine_ref[...]
        mb = jax.lax.bitcast_convert_type(mall, jnp.int32)
        kcol = kv_ref[:, 0:1]

        def _bisect(_, carry):
            lo, hi = carry
            mid = lo + (hi - lo) // 2
            cnt = jnp.sum((mb >= mid).astype(jnp.float32),
                          axis=1, keepdims=True)
            gek = cnt >= kcol
            return (jnp.where(gek, mid, lo), jnp.where(gek, hi, mid))

        lo0 = jnp.zeros((_B, 1), jnp.int32)
        hi0 = jnp.full((_B, 1), 0x7F800000, jnp.int32)
        lo, _hi = jax.lax.fori_loop(0, 31, _bisect, (lo0, hi0))
        gt = mb > lo
        count_gt = jnp.sum(gt.astype(jnp.float32), axis=1, keepdims=True)
        sum_gt = jnp.sum(jnp.where(gt, mall, 0.0), axis=1, keepdims=True)
        tval = jnp.max(jnp.where(mb <= lo, mall, -1.0), axis=1, keepdims=True)
        neg = jnp.where(kcol > 0.0, sum_gt + (kcol - count_gt) * tval, 0.0)
        lc_ref[...] += jnp.full((1, 1), jnp.sum(neg), jnp.float32)


@jax.jit
def kernel(loc_data, conf_data, priors, targets):
    pp = _P_PAD - _P
    confT = jnp.pad(conf_data, ((0, 0), (0, pp), (0, _C_PAD - _C)),
                    constant_values=-1e30)
    locT = jnp.pad(loc_data.transpose(0, 2, 1), ((0, 0), (0, 0), (0, pp)))
    priorsT = jnp.pad(priors.T, ((0, 0), (0, pp)))
    truths8 = jnp.pad(targets, ((0, 0), (0, 0), (0, 3)))
    truthsT8 = jnp.pad(targets.transpose(0, 2, 1), ((0, 0), (0, 3), (0, 0)))

    ll, lc, n = pl.pallas_call(
        _body,
        grid=(_B,),
        in_specs=[
            pl.BlockSpec((1, _O, 8), lambda i: (i, 0, 0)),
            pl.BlockSpec((1, 8, _O), lambda i: (i, 0, 0)),
            pl.BlockSpec((4, _P_PAD), lambda i: (0, 0)),
            pl.BlockSpec((1, 4, _P_PAD), lambda i: (i, 0, 0)),
            pl.BlockSpec((1, _P_PAD, _C_PAD), lambda i: (i, 0, 0)),
        ],
        out_specs=[
            pl.BlockSpec((1, 1), lambda i: (0, 0)),
            pl.BlockSpec((1, 1), lambda i: (0, 0)),
            pl.BlockSpec((1, 1), lambda i: (0, 0)),
        ],
        out_shape=[
            jax.ShapeDtypeStruct((1, 1), jnp.float32),
            jax.ShapeDtypeStruct((1, 1), jnp.float32),
            jax.ShapeDtypeStruct((1, 1), jnp.float32),
        ],
        scratch_shapes=[
            pltpu.VMEM((_B, _P_PAD), jnp.float32),
            pltpu.VMEM((_B, 128), jnp.float32),
        ],
    )(truths8, truthsT8, priorsT, locT, confT)

    N = n[0, 0]
    return ll[0, 0] / N, lc[0, 0] / N


# speedup vs baseline: 1.8885x; 1.8885x over previous
import jax
import jax.numpy as jnp
from jax.experimental import pallas as pl
from jax.experimental.pallas import tpu as pltpu

_B = 32
_P = 8732
_C = 21
_O = 16
_P_PAD = 8960
_C_PAD = 24
_NEG_POS = 3
_THRESHOLD = 0.5
_V0 = 0.1
_V1 = 0.2


def _body(truths_ref, truthsT_ref, pf_ref, locT_ref, confT_ref,
          ll_ref, lc_ref, n_ref, mine_ref, kv_ref):
    b = pl.program_id(0)

    t = truths_ref[0]
    tx1 = t[:, 0:1]
    ty1 = t[:, 1:2]
    tx2 = t[:, 2:3]
    ty2 = t[:, 3:4]

    pf = pf_ref[...]
    px1 = pf[0:1]
    py1 = pf[1:2]
    px2 = pf[2:3]
    py2 = pf[3:4]
    area_p = pf[4:5]
    pcx = pf[5:6]
    pcy = pf[6:7]
    inv_v0w = pf[7:8]
    inv_v0h = pf[8:9]
    log_pw = pf[9:10]
    log_ph = pf[10:11]

    iw = jnp.maximum(jnp.minimum(tx2, px2) - jnp.maximum(tx1, px1), 0.0)
    ih = jnp.maximum(jnp.minimum(ty2, py2) - jnp.maximum(ty1, py1), 0.0)
    inter = iw * ih
    area_t = (tx2 - tx1) * (ty2 - ty1)
    ov = inter / (area_t + area_p - inter)

    o_iota = jax.lax.broadcasted_iota(jnp.int32, (_O, _P_PAD), 0)
    p_iota = jax.lax.broadcasted_iota(jnp.int32, (_O, _P_PAD), 1)

    bto = jnp.max(ov, axis=0, keepdims=True)
    bti = jnp.min(jnp.where(ov == bto, o_iota, _O), axis=0, keepdims=True)

    bpo = jnp.max(ov, axis=1, keepdims=True)
    bpi = jnp.min(jnp.where(ov == bpo, p_iota, _P_PAD), axis=1, keepdims=True)

    over_idx = jnp.max(jnp.where(p_iota == bpi, o_iota, -1),
                       axis=0, keepdims=True)
    any_hit = over_idx >= 0
    bto = jnp.where(any_hit, 2.0, bto)
    bti = jnp.where(any_hit, over_idx, bti)

    sel = (o_iota == bti).astype(jnp.float32)
    tT = truthsT_ref[0]
    mm = jax.lax.dot_general(tT, sel, (((1,), (0,)), ((), ())),
                             precision=jax.lax.Precision.HIGHEST,
                             preferred_element_type=jnp.float32)
    mx1 = mm[0:1]
    my1 = mm[1:2]
    mx2 = mm[2:3]
    my2 = mm[3:4]
    mlab = mm[4:5]

    conf_t = jnp.where(bto < _THRESHOLD, 0.0, mlab + 1.0)
    pos = conf_t > 0.0

    g_cx = ((mx1 + mx2) * 0.5 - pcx) * inv_v0w
    g_cy = ((my1 + my2) * 0.5 - pcy) * inv_v0h
    g_w = (jnp.log(mx2 - mx1) - log_pw) * (1.0 / _V1)
    g_h = (jnp.log(my2 - my1) - log_ph) * (1.0 / _V1)

    ld = locT_ref[0]

    def _sl1(d):
        a = jnp.abs(d)
        c = jnp.minimum(a, 1.0)
        return c * (a - 0.5 * c)

    sl1 = (_sl1(ld[0:1] - g_cx) + _sl1(ld[1:2] - g_cy)
           + _sl1(ld[2:3] - g_w) + _sl1(ld[3:4] - g_h))
    loss_l_row = jnp.sum(jnp.where(pos, sl1, 0.0))

    cf = confT_ref[0]
    mrow = jnp.max(cf, axis=0, keepdims=True)
    ex = jnp.exp(cf - mrow)
    c_iota = jax.lax.broadcasted_iota(jnp.int32, (_C_PAD, _P_PAD), 0)
    conf_t_i = conf_t.astype(jnp.int32)
    gsel = jnp.where(c_iota == conf_t_i, cf, 0.0)
    ones_row = jnp.ones((1, _C_PAD), jnp.float32)
    s = jax.lax.dot_general(ones_row, ex, (((1,), (0,)), ((), ())),
                            precision=jax.lax.Precision.HIGHEST,
                            preferred_element_type=jnp.float32)
    gathered = jax.lax.dot_general(ones_row, gsel, (((1,), (0,)), ((), ())),
                                   precision=jax.lax.Precision.HIGHEST,
                                   preferred_element_type=jnp.float32)
    ce = jnp.log(s) + mrow - gathered

    valid = jax.lax.broadcasted_iota(jnp.int32, (1, _P_PAD), 1) < _P
    mining = jnp.where(pos | (~valid), 0.0, ce)
    ce_pos = jnp.sum(jnp.where(pos, ce, 0.0))
    npos = jnp.sum(pos.astype(jnp.float32))
    k = jnp.minimum(_NEG_POS * npos, float(_P - 1))

    mine_ref[pl.ds(b, 1), :] = mining
    kv_ref[pl.ds(b, 1), :] = jnp.full((1, 128), k, jnp.float32)

    @pl.when(b == 0)
    def _init():
        ll_ref[...] = jnp.zeros((1, 1), jnp.float32)
        lc_ref[...] = jnp.zeros((1, 1), jnp.float32)
        n_ref[...] = jnp.zeros((1, 1), jnp.float32)

    ll_ref[...] += jnp.full((1, 1), loss_l_row, jnp.float32)
    lc_ref[...] += jnp.full((1, 1), ce_pos, jnp.float32)
    n_ref[...] += jnp.full((1, 1), npos, jnp.float32)

    @pl.when(b == _B - 1)
    def _mine():
        mall = mine_ref[...]
        mb = jax.lax.bitcast_convert_type(mall, jnp.int32)
        kcol = kv_ref[:, 0:1]

        def _bisect(_, carry):
            lo, hi = carry
            mid = lo + (hi - lo) // 2
            cnt = jnp.sum((mb >= mid).astype(jnp.float32),
                          axis=1, keepdims=True)
            gek = cnt >= kcol
            return (jnp.where(gek, mid, lo), jnp.where(gek, hi, mid))

        lo0 = jnp.zeros((_B, 1), jnp.int32)
        hi0 = jnp.full((_B, 1), 0x7F800000, jnp.int32)
        lo, _hi = jax.lax.fori_loop(0, 31, _bisect, (lo0, hi0))
        gt = mb > lo
        count_gt = jnp.sum(gt.astype(jnp.float32), axis=1, keepdims=True)
        sum_gt = jnp.sum(jnp.where(gt, mall, 0.0), axis=1, keepdims=True)
        tval = jnp.max(jnp.where(mb <= lo, mall, -1.0), axis=1, keepdims=True)
        neg = jnp.where(kcol > 0.0, sum_gt + (kcol - count_gt) * tval, 0.0)
        lc_ref[...] += jnp.full((1, 1), jnp.sum(neg), jnp.float32)


@jax.jit
def kernel(loc_data, conf_data, priors, targets):
    pp = _P_PAD - _P
    confT = jnp.pad(conf_data.transpose(0, 2, 1),
                    ((0, 0), (0, _C_PAD - _C), (0, pp)),
                    constant_values=-1e30)
    locT = jnp.pad(loc_data.transpose(0, 2, 1), ((0, 0), (0, 0), (0, pp)))
    truths8 = jnp.pad(targets, ((0, 0), (0, 0), (0, 3)))
    truthsT8 = jnp.pad(targets.transpose(0, 2, 1), ((0, 0), (0, 3), (0, 0)))

    pcx, pcy, pw, ph = priors[:, 0], priors[:, 1], priors[:, 2], priors[:, 3]
    px1 = pcx - pw / 2.0
    py1 = pcy - ph / 2.0
    px2 = pcx + pw / 2.0
    py2 = pcy + ph / 2.0
    feats = jnp.stack([
        px1, py1, px2, py2,
        (px2 - px1) * (py2 - py1), pcx, pcy,
        1.0 / (_V0 * pw), 1.0 / (_V0 * ph),
        jnp.log(pw), jnp.log(ph),
    ], axis=0)
    pfeat = jnp.pad(feats, ((0, 16 - feats.shape[0]), (0, pp)))

    ll, lc, n = pl.pallas_call(
        _body,
        grid=(_B,),
        in_specs=[
            pl.BlockSpec((1, _O, 8), lambda i: (i, 0, 0)),
            pl.BlockSpec((1, 8, _O), lambda i: (i, 0, 0)),
            pl.BlockSpec((16, _P_PAD), lambda i: (0, 0)),
            pl.BlockSpec((1, 4, _P_PAD), lambda i: (i, 0, 0)),
            pl.BlockSpec((1, _C_PAD, _P_PAD), lambda i: (i, 0, 0)),
        ],
        out_specs=[
            pl.BlockSpec((1, 1), lambda i: (0, 0)),
            pl.BlockSpec((1, 1), lambda i: (0, 0)),
            pl.BlockSpec((1, 1), lambda i: (0, 0)),
        ],
        out_shape=[
            jax.ShapeDtypeStruct((1, 1), jnp.float32),
            jax.ShapeDtypeStruct((1, 1), jnp.float32),
            jax.ShapeDtypeStruct((1, 1), jnp.float32),
        ],
        scratch_shapes=[
            pltpu.VMEM((_B, _P_PAD), jnp.float32),
            pltpu.VMEM((_B, 128), jnp.float32),
        ],
    )(truths8, truthsT8, pfeat, locT, confT)

    N = n[0, 0]
    return ll[0, 0] / N, lc[0, 0] / N


# speedup vs baseline: 2.2115x; 1.1710x over previous
import jax
import jax.numpy as jnp
from jax.experimental import pallas as pl
from jax.experimental.pallas import tpu as pltpu

_B = 32
_P = 8732
_C = 21
_O = 16
_P_PAD = 8960
_C_PAD = 24
_NEG_POS = 3
_THRESHOLD = 0.5
_V0 = 0.1
_V1 = 0.2


def _body(truths_ref, truthsT_ref, pf_ref, locT_ref, confT_ref,
          ll_ref, lc_ref, n_ref, mine_ref, kv_ref):
    b = pl.program_id(0)

    t = truths_ref[0]
    tx1 = t[:, 0:1]
    ty1 = t[:, 1:2]
    tx2 = t[:, 2:3]
    ty2 = t[:, 3:4]

    pf = pf_ref[...]
    px1 = pf[0:1]
    py1 = pf[1:2]
    px2 = pf[2:3]
    py2 = pf[3:4]
    area_p = pf[4:5]
    pcx = pf[5:6]
    pcy = pf[6:7]
    inv_v0w = pf[7:8]
    inv_v0h = pf[8:9]
    log_pw = pf[9:10]
    log_ph = pf[10:11]

    iw = jnp.maximum(jnp.minimum(tx2, px2) - jnp.maximum(tx1, px1), 0.0)
    ih = jnp.maximum(jnp.minimum(ty2, py2) - jnp.maximum(ty1, py1), 0.0)
    inter = iw * ih
    area_t = (tx2 - tx1) * (ty2 - ty1)
    ov = inter / (area_t + area_p - inter)

    o_iota = jax.lax.broadcasted_iota(jnp.int32, (_O, _P_PAD), 0)
    p_iota = jax.lax.broadcasted_iota(jnp.int32, (_O, _P_PAD), 1)

    bto = jnp.max(ov, axis=0, keepdims=True)
    bti = jnp.min(jnp.where(ov == bto, o_iota, _O), axis=0, keepdims=True)

    bpo = jnp.max(ov, axis=1, keepdims=True)
    bpi = jnp.min(jnp.where(ov == bpo, p_iota, _P_PAD), axis=1, keepdims=True)

    over_idx = jnp.max(jnp.where(p_iota == bpi, o_iota, -1),
                       axis=0, keepdims=True)
    any_hit = over_idx >= 0
    bto = jnp.where(any_hit, 2.0, bto)
    bti = jnp.where(any_hit, over_idx, bti)

    sel = (o_iota == bti).astype(jnp.float32)
    tT = truthsT_ref[0]
    mm = jax.lax.dot_general(tT, sel, (((1,), (0,)), ((), ())),
                             precision=jax.lax.Precision.HIGHEST,
                             preferred_element_type=jnp.float32)
    mx1 = mm[0:1]
    my1 = mm[1:2]
    mx2 = mm[2:3]
    my2 = mm[3:4]
    mlab = mm[4:5]

    conf_t = jnp.where(bto < _THRESHOLD, 0.0, mlab + 1.0)
    pos = conf_t > 0.0

    g_cx = ((mx1 + mx2) * 0.5 - pcx) * inv_v0w
    g_cy = ((my1 + my2) * 0.5 - pcy) * inv_v0h
    g_w = (jnp.log(mx2 - mx1) - log_pw) * (1.0 / _V1)
    g_h = (jnp.log(my2 - my1) - log_ph) * (1.0 / _V1)

    ld = locT_ref[0]

    def _sl1(d):
        a = jnp.abs(d)
        c = jnp.minimum(a, 1.0)
        return c * (a - 0.5 * c)

    sl1 = (_sl1(ld[0:1] - g_cx) + _sl1(ld[1:2] - g_cy)
           + _sl1(ld[2:3] - g_w) + _sl1(ld[3:4] - g_h))
    loss_l_row = jnp.sum(jnp.where(pos, sl1, 0.0))

    cf = confT_ref[0]
    mrow = jnp.max(cf, axis=0, keepdims=True)
    ex = jnp.exp(cf - mrow)
    c_iota = jax.lax.broadcasted_iota(jnp.int32, (_C_PAD, _P_PAD), 0)
    conf_t_i = conf_t.astype(jnp.int32)
    gsel = jnp.where(c_iota == conf_t_i, cf, 0.0)
    s = jnp.sum(ex, axis=0, keepdims=True)
    gathered = jnp.sum(gsel, axis=0, keepdims=True)
    ce = jnp.log(s) + mrow - gathered

    valid = jax.lax.broadcasted_iota(jnp.int32, (1, _P_PAD), 1) < _P
    mining = jnp.where(pos | (~valid), 0.0, ce)
    ce_pos = jnp.sum(jnp.where(pos, ce, 0.0))
    npos = jnp.sum(pos.astype(jnp.float32))
    k = jnp.minimum(_NEG_POS * npos, float(_P - 1))

    mine_ref[pl.ds(b, 1), :] = mining
    kv_ref[pl.ds(b, 1), :] = jnp.full((1, 128), k, jnp.float32)

    @pl.when(b == 0)
    def _init():
        ll_ref[...] = jnp.zeros((1, 1), jnp.float32)
        lc_ref[...] = jnp.zeros((1, 1), jnp.float32)
        n_ref[...] = jnp.zeros((1, 1), jnp.float32)

    ll_ref[...] += jnp.full((1, 1), loss_l_row, jnp.float32)
    lc_ref[...] += jnp.full((1, 1), ce_pos, jnp.float32)
    n_ref[...] += jnp.full((1, 1), npos, jnp.float32)

    @pl.when(b == _B - 1)
    def _mine():
        mall = mine_ref[...]
        mb = jax.lax.bitcast_convert_type(mall, jnp.int32)
        kcol = kv_ref[:, 0:1]

        def _bisect(_, carry):
            lo, hi = carry
            mid = lo + (hi - lo) // 2
            cnt = jnp.sum((mb >= mid).astype(jnp.float32),
                          axis=1, keepdims=True)
            gek = cnt >= kcol
            return (jnp.where(gek, mid, lo), jnp.where(gek, hi, mid))

        lo0 = jnp.zeros((_B, 1), jnp.int32)
        hi0 = jnp.full((_B, 1), 0x7F800000, jnp.int32)
        lo, _hi = jax.lax.fori_loop(0, 31, _bisect, (lo0, hi0))
        gt = mb > lo
        count_gt = jnp.sum(gt.astype(jnp.float32), axis=1, keepdims=True)
        sum_gt = jnp.sum(jnp.where(gt, mall, 0.0), axis=1, keepdims=True)
        tval = jnp.max(jnp.where(mb <= lo, mall, -1.0), axis=1, keepdims=True)
        neg = jnp.where(kcol > 0.0, sum_gt + (kcol - count_gt) * tval, 0.0)
        lc_ref[...] += jnp.full((1, 1), jnp.sum(neg), jnp.float32)


@jax.jit
def kernel(loc_data, conf_data, priors, targets):
    pp = _P_PAD - _P
    confT = jnp.pad(conf_data.transpose(0, 2, 1),
                    ((0, 0), (0, _C_PAD - _C), (0, pp)),
                    constant_values=-1e30)
    locT = jnp.pad(loc_data.transpose(0, 2, 1), ((0, 0), (0, 0), (0, pp)))
    truths8 = jnp.pad(targets, ((0, 0), (0, 0), (0, 3)))
    truthsT8 = jnp.pad(targets.transpose(0, 2, 1), ((0, 0), (0, 3), (0, 0)))

    pcx, pcy, pw, ph = priors[:, 0], priors[:, 1], priors[:, 2], priors[:, 3]
    px1 = pcx - pw / 2.0
    py1 = pcy - ph / 2.0
    px2 = pcx + pw / 2.0
    py2 = pcy + ph / 2.0
    feats = jnp.stack([
        px1, py1, px2, py2,
        (px2 - px1) * (py2 - py1), pcx, pcy,
        1.0 / (_V0 * pw), 1.0 / (_V0 * ph),
        jnp.log(pw), jnp.log(ph),
    ], axis=0)
    pfeat = jnp.pad(feats, ((0, 16 - feats.shape[0]), (0, pp)))

    ll, lc, n = pl.pallas_call(
        _body,
        grid=(_B,),
        in_specs=[
            pl.BlockSpec((1, _O, 8), lambda i: (i, 0, 0)),
            pl.BlockSpec((1, 8, _O), lambda i: (i, 0, 0)),
            pl.BlockSpec((16, _P_PAD), lambda i: (0, 0)),
            pl.BlockSpec((1, 4, _P_PAD), lambda i: (i, 0, 0)),
            pl.BlockSpec((1, _C_PAD, _P_PAD), lambda i: (i, 0, 0)),
        ],
        out_specs=[
            pl.BlockSpec((1, 1), lambda i: (0, 0)),
            pl.BlockSpec((1, 1), lambda i: (0, 0)),
            pl.BlockSpec((1, 1), lambda i: (0, 0)),
        ],
        out_shape=[
            jax.ShapeDtypeStruct((1, 1), jnp.float32),
            jax.ShapeDtypeStruct((1, 1), jnp.float32),
            jax.ShapeDtypeStruct((1, 1), jnp.float32),
        ],
        scratch_shapes=[
            pltpu.VMEM((_B, _P_PAD), jnp.float32),
            pltpu.VMEM((_B, 128), jnp.float32),
        ],
    )(truths8, truthsT8, pfeat, locT, confT)

    N = n[0, 0]
    return ll[0, 0] / N, lc[0, 0] / N


# speedup vs baseline: 2.4256x; 1.0968x over previous
import jax
import jax.numpy as jnp
from jax.experimental import pallas as pl
from jax.experimental.pallas import tpu as pltpu

_B = 32
_P = 8732
_C = 21
_O = 16
_P_PAD = 8960
_C_PAD = 24
_NEG_POS = 3
_THRESHOLD = 0.5
_V0 = 0.1
_V1 = 0.2


def _body(truths_ref, truthsT_ref, pf_ref, locT_ref, confT_ref,
          ll_ref, lc_ref, n_ref, mine_ref, kv_ref):
    b = pl.program_id(0)

    t = truths_ref[0]
    tx1 = t[:, 0:1]
    ty1 = t[:, 1:2]
    tx2 = t[:, 2:3]
    ty2 = t[:, 3:4]

    pf = pf_ref[...]
    px1 = pf[0:1]
    py1 = pf[1:2]
    px2 = pf[2:3]
    py2 = pf[3:4]
    area_p = pf[4:5]
    pcx = pf[5:6]
    pcy = pf[6:7]
    inv_v0w = pf[7:8]
    inv_v0h = pf[8:9]
    log_pw = pf[9:10]
    log_ph = pf[10:11]

    iw = jnp.maximum(jnp.minimum(tx2, px2) - jnp.maximum(tx1, px1), 0.0)
    ih = jnp.maximum(jnp.minimum(ty2, py2) - jnp.maximum(ty1, py1), 0.0)
    inter = iw * ih
    area_t = (tx2 - tx1) * (ty2 - ty1)
    ov = inter / (area_t + area_p - inter)

    o_iota = jax.lax.broadcasted_iota(jnp.int32, (_O, _P_PAD), 0)
    p_iota = jax.lax.broadcasted_iota(jnp.int32, (_O, _P_PAD), 1)

    bto = jnp.max(ov, axis=0, keepdims=True)
    bti = jnp.min(jnp.where(ov == bto, o_iota, _O), axis=0, keepdims=True)

    bpo = jnp.max(ov, axis=1, keepdims=True)
    bpi = jnp.min(jnp.where(ov == bpo, p_iota, _P_PAD), axis=1, keepdims=True)

    over_idx = jnp.max(jnp.where(p_iota == bpi, o_iota, -1),
                       axis=0, keepdims=True)
    any_hit = over_idx >= 0
    bto = jnp.where(any_hit, 2.0, bto)
    bti = jnp.where(any_hit, over_idx, bti)

    sel = (o_iota == bti).astype(jnp.bfloat16)
    tT = truthsT_ref[0]
    tT_hi = tT.astype(jnp.bfloat16)
    tT_lo = (tT - tT_hi.astype(jnp.float32)).astype(jnp.bfloat16)
    dn = (((1,), (0,)), ((), ()))
    mm = (jax.lax.dot_general(tT_hi, sel, dn,
                              preferred_element_type=jnp.float32)
          + jax.lax.dot_general(tT_lo, sel, dn,
                                preferred_element_type=jnp.float32))
    mx1 = mm[0:1]
    my1 = mm[1:2]
    mx2 = mm[2:3]
    my2 = mm[3:4]
    mlab = mm[4:5]

    conf_t = jnp.where(bto < _THRESHOLD, 0.0, mlab + 1.0)
    pos = conf_t > 0.0

    g_cx = ((mx1 + mx2) * 0.5 - pcx) * inv_v0w
    g_cy = ((my1 + my2) * 0.5 - pcy) * inv_v0h
    g_w = (jnp.log(mx2 - mx1) - log_pw) * (1.0 / _V1)
    g_h = (jnp.log(my2 - my1) - log_ph) * (1.0 / _V1)

    ld = locT_ref[0]

    def _sl1(d):
        a = jnp.abs(d)
        c = jnp.minimum(a, 1.0)
        return c * (a - 0.5 * c)

    sl1 = (_sl1(ld[0:1] - g_cx) + _sl1(ld[1:2] - g_cy)
           + _sl1(ld[2:3] - g_w) + _sl1(ld[3:4] - g_h))
    loss_l_row = jnp.sum(jnp.where(pos, sl1, 0.0))

    cf = confT_ref[0].astype(jnp.float32)
    mrow = jnp.max(cf, axis=0, keepdims=True)
    ex = jnp.exp(cf - mrow)
    c_iota = jax.lax.broadcasted_iota(jnp.int32, (_C_PAD, _P_PAD), 0)
    conf_t_i = conf_t.astype(jnp.int32)
    gsel = jnp.where(c_iota == conf_t_i, cf, 0.0)
    s = jnp.sum(ex, axis=0, keepdims=True)
    gathered = jnp.sum(gsel, axis=0, keepdims=True)
    ce = jnp.log(s) + mrow - gathered

    valid = jax.lax.broadcasted_iota(jnp.int32, (1, _P_PAD), 1) < _P
    mining = jnp.where(pos | (~valid), 0.0, ce)
    ce_pos = jnp.sum(jnp.where(pos, ce, 0.0))
    npos = jnp.sum(pos.astype(jnp.float32))
    k = jnp.minimum(_NEG_POS * npos, float(_P - 1))

    mine_ref[pl.ds(b, 1), :] = mining
    kv_ref[pl.ds(b, 1), :] = jnp.full((1, 128), k, jnp.float32)

    @pl.when(b == 0)
    def _init():
        ll_ref[...] = jnp.zeros((1, 1), jnp.float32)
        lc_ref[...] = jnp.zeros((1, 1), jnp.float32)
        n_ref[...] = jnp.zeros((1, 1), jnp.float32)

    ll_ref[...] += jnp.full((1, 1), loss_l_row, jnp.float32)
    lc_ref[...] += jnp.full((1, 1), ce_pos, jnp.float32)
    n_ref[...] += jnp.full((1, 1), npos, jnp.float32)

    @pl.when(b == _B - 1)
    def _mine():
        mall = mine_ref[...]
        mb = jax.lax.bitcast_convert_type(mall, jnp.int32)
        kcol = kv_ref[:, 0:1]

        def _bisect(_, carry):
            lo, hi = carry
            mid = lo + (hi - lo) // 2
            cnt = jnp.sum((mb >= mid).astype(jnp.float32),
                          axis=1, keepdims=True)
            gek = cnt >= kcol
            return (jnp.where(gek, mid, lo), jnp.where(gek, hi, mid))

        lo0 = jnp.zeros((_B, 1), jnp.int32)
        hi0 = jnp.full((_B, 1), 0x7F800000, jnp.int32)
        lo, _hi = jax.lax.fori_loop(0, 31, _bisect, (lo0, hi0))
        gt = mb > lo
        count_gt = jnp.sum(gt.astype(jnp.float32), axis=1, keepdims=True)
        sum_gt = jnp.sum(jnp.where(gt, mall, 0.0), axis=1, keepdims=True)
        tval = jnp.max(jnp.where(mb <= lo, mall, -1.0), axis=1, keepdims=True)
        neg = jnp.where(kcol > 0.0, sum_gt + (kcol - count_gt) * tval, 0.0)
        lc_ref[...] += jnp.full((1, 1), jnp.sum(neg), jnp.float32)


@jax.jit
def kernel(loc_data, conf_data, priors, targets):
    pp = _P_PAD - _P
    confT = jnp.pad(conf_data.astype(jnp.bfloat16).transpose(0, 2, 1),
                    ((0, 0), (0, _C_PAD - _C), (0, pp)),
                    constant_values=jnp.bfloat16(-1e30))
    locT = jnp.pad(loc_data.transpose(0, 2, 1), ((0, 0), (0, 0), (0, pp)))
    truths8 = jnp.pad(targets, ((0, 0), (0, 0), (0, 3)))
    truthsT8 = jnp.pad(targets.transpose(0, 2, 1), ((0, 0), (0, 3), (0, 0)))

    pcx, pcy, pw, ph = priors[:, 0], priors[:, 1], priors[:, 2], priors[:, 3]
    px1 = pcx - pw / 2.0
    py1 = pcy - ph / 2.0
    px2 = pcx + pw / 2.0
    py2 = pcy + ph / 2.0
    feats = jnp.stack([
        px1, py1, px2, py2,
        (px2 - px1) * (py2 - py1), pcx, pcy,
        1.0 / (_V0 * pw), 1.0 / (_V0 * ph),
        jnp.log(pw), jnp.log(ph),
    ], axis=0)
    pfeat = jnp.pad(feats, ((0, 16 - feats.shape[0]), (0, pp)))

    ll, lc, n = pl.pallas_call(
        _body,
        grid=(_B,),
        in_specs=[
            pl.BlockSpec((1, _O, 8), lambda i: (i, 0, 0)),
            pl.BlockSpec((1, 8, _O), lambda i: (i, 0, 0)),
            pl.BlockSpec((16, _P_PAD), lambda i: (0, 0)),
            pl.BlockSpec((1, 4, _P_PAD), lambda i: (i, 0, 0)),
            pl.BlockSpec((1, _C_PAD, _P_PAD), lambda i: (i, 0, 0)),
        ],
        out_specs=[
            pl.BlockSpec((1, 1), lambda i: (0, 0)),
            pl.BlockSpec((1, 1), lambda i: (0, 0)),
            pl.BlockSpec((1, 1), lambda i: (0, 0)),
        ],
        out_shape=[
            jax.ShapeDtypeStruct((1, 1), jnp.float32),
            jax.ShapeDtypeStruct((1, 1), jnp.float32),
            jax.ShapeDtypeStruct((1, 1), jnp.float32),
        ],
        scratch_shapes=[
            pltpu.VMEM((_B, _P_PAD), jnp.float32),
            pltpu.VMEM((_B, 128), jnp.float32),
        ],
    )(truths8, truthsT8, pfeat, locT, confT)

    N = n[0, 0]
    return ll[0, 0] / N, lc[0, 0] / N


# speedup vs baseline: 2.4304x; 1.0020x over previous
import jax
import jax.numpy as jnp
from jax.experimental import pallas as pl
from jax.experimental.pallas import tpu as pltpu

_B = 32
_P = 8732
_C = 21
_O = 16
_P_PAD = 8960
_C_PAD = 24
_NEG_POS = 3
_THRESHOLD = 0.5
_V0 = 0.1
_V1 = 0.2


def _body(truths_ref, truthsT_ref, pf_ref, locT_ref, confT_ref,
          ll_ref, lc_ref, n_ref, mine_ref, kv_ref):
    b = pl.program_id(0)

    t = truths_ref[0]
    tx1 = t[:, 0:1]
    ty1 = t[:, 1:2]
    tx2 = t[:, 2:3]
    ty2 = t[:, 3:4]

    pf = pf_ref[...]
    px1 = pf[0:1]
    py1 = pf[1:2]
    px2 = pf[2:3]
    py2 = pf[3:4]
    area_p = pf[4:5]
    pcx = pf[5:6]
    pcy = pf[6:7]
    inv_v0w = pf[7:8]
    inv_v0h = pf[8:9]
    log_pw = pf[9:10]
    log_ph = pf[10:11]

    iw = jnp.maximum(jnp.minimum(tx2, px2) - jnp.maximum(tx1, px1), 0.0)
    ih = jnp.maximum(jnp.minimum(ty2, py2) - jnp.maximum(ty1, py1), 0.0)
    inter = iw * ih
    area_t = (tx2 - tx1) * (ty2 - ty1)
    ov = inter / (area_t + area_p - inter)

    o_iota = jax.lax.broadcasted_iota(jnp.int32, (_O, _P_PAD), 0)
    p_iota = jax.lax.broadcasted_iota(jnp.int32, (_O, _P_PAD), 1)

    bto = jnp.max(ov, axis=0, keepdims=True)
    bti = jnp.min(jnp.where(ov == bto, o_iota, _O), axis=0, keepdims=True)

    bpo = jnp.max(ov, axis=1, keepdims=True)
    bpi = jnp.min(jnp.where(ov == bpo, p_iota, _P_PAD), axis=1, keepdims=True)

    over_idx = jnp.max(jnp.where(p_iota == bpi, o_iota, -1),
                       axis=0, keepdims=True)
    any_hit = over_idx >= 0
    bto = jnp.where(any_hit, 2.0, bto)
    bti = jnp.where(any_hit, over_idx, bti)

    sel = (o_iota == bti).astype(jnp.bfloat16)
    tT = truthsT_ref[0]
    tT_hi = tT.astype(jnp.bfloat16)
    tT_lo = (tT - tT_hi.astype(jnp.float32)).astype(jnp.bfloat16)
    dn = (((1,), (0,)), ((), ()))
    mm = (jax.lax.dot_general(tT_hi, sel, dn,
                              preferred_element_type=jnp.float32)
          + jax.lax.dot_general(tT_lo, sel, dn,
                                preferred_element_type=jnp.float32))
    mx1 = mm[0:1]
    my1 = mm[1:2]
    mx2 = mm[2:3]
    my2 = mm[3:4]
    mlab = mm[4:5]

    conf_t = jnp.where(bto < _THRESHOLD, 0.0, mlab + 1.0)
    pos = conf_t > 0.0

    g_cx = ((mx1 + mx2) * 0.5 - pcx) * inv_v0w
    g_cy = ((my1 + my2) * 0.5 - pcy) * inv_v0h
    g_w = (jnp.log(mx2 - mx1) - log_pw) * (1.0 / _V1)
    g_h = (jnp.log(my2 - my1) - log_ph) * (1.0 / _V1)

    ld = locT_ref[0].astype(jnp.float32)

    def _sl1(d):
        a = jnp.abs(d)
        c = jnp.minimum(a, 1.0)
        return c * (a - 0.5 * c)

    sl1 = (_sl1(ld[0:1] - g_cx) + _sl1(ld[1:2] - g_cy)
           + _sl1(ld[2:3] - g_w) + _sl1(ld[3:4] - g_h))
    loss_l_row = jnp.sum(jnp.where(pos, sl1, 0.0))

    cf = confT_ref[0].astype(jnp.float32)
    mrow = jnp.max(cf, axis=0, keepdims=True)
    ex = jnp.exp(cf - mrow)
    c_iota = jax.lax.broadcasted_iota(jnp.int32, (_C_PAD, _P_PAD), 0)
    conf_t_i = conf_t.astype(jnp.int32)
    gsel = jnp.where(c_iota == conf_t_i, cf, 0.0)
    s = jnp.sum(ex, axis=0, keepdims=True)
    gathered = jnp.sum(gsel, axis=0, keepdims=True)
    ce = jnp.log(s) + mrow - gathered

    valid = jax.lax.broadcasted_iota(jnp.int32, (1, _P_PAD), 1) < _P
    mining = jnp.where(pos | (~valid), 0.0, ce)
    ce_pos = jnp.sum(jnp.where(pos, ce, 0.0))
    npos = jnp.sum(pos.astype(jnp.float32))
    k = jnp.minimum(_NEG_POS * npos, float(_P - 1))

    mine_ref[pl.ds(b, 1), :] = mining
    kv_ref[pl.ds(b, 1), :] = jnp.full((1, 128), k, jnp.float32)

    @pl.when(b == 0)
    def _init():
        ll_ref[...] = jnp.zeros((1, 1), jnp.float32)
        lc_ref[...] = jnp.zeros((1, 1), jnp.float32)
        n_ref[...] = jnp.zeros((1, 1), jnp.float32)

    ll_ref[...] += jnp.full((1, 1), loss_l_row, jnp.float32)
    lc_ref[...] += jnp.full((1, 1), ce_pos, jnp.float32)
    n_ref[...] += jnp.full((1, 1), npos, jnp.float32)

    @pl.when(b == _B - 1)
    def _mine():
        mall = mine_ref[...]
        mb = jax.lax.bitcast_convert_type(mall, jnp.int32)
        kcol = kv_ref[:, 0:1]

        def _bisect(_, carry):
            lo, hi = carry
            mid = lo + (hi - lo) // 2
            cnt = jnp.sum((mb >= mid).astype(jnp.float32),
                          axis=1, keepdims=True)
            gek = cnt >= kcol
            return (jnp.where(gek, mid, lo), jnp.where(gek, hi, mid))

        lo0 = jnp.zeros((_B, 1), jnp.int32)
        hi0 = jnp.full((_B, 1), 0x7F800000, jnp.int32)
        lo, _hi = jax.lax.fori_loop(0, 31, _bisect, (lo0, hi0))
        gt = mb > lo
        count_gt = jnp.sum(gt.astype(jnp.float32), axis=1, keepdims=True)
        sum_gt = jnp.sum(jnp.where(gt, mall, 0.0), axis=1, keepdims=True)
        tval = jnp.max(jnp.where(mb <= lo, mall, -1.0), axis=1, keepdims=True)
        neg = jnp.where(kcol > 0.0, sum_gt + (kcol - count_gt) * tval, 0.0)
        lc_ref[...] += jnp.full((1, 1), jnp.sum(neg), jnp.float32)


@jax.jit
def kernel(loc_data, conf_data, priors, targets):
    pp = _P_PAD - _P
    confT = jnp.pad(conf_data.astype(jnp.bfloat16).transpose(0, 2, 1),
                    ((0, 0), (0, _C_PAD - _C), (0, pp)),
                    constant_values=jnp.bfloat16(-1e30))
    locT = jnp.pad(loc_data.astype(jnp.bfloat16).transpose(0, 2, 1),
                   ((0, 0), (0, 0), (0, pp)))
    truths8 = jnp.pad(targets, ((0, 0), (0, 0), (0, 3)))
    truthsT8 = jnp.pad(targets.transpose(0, 2, 1), ((0, 0), (0, 3), (0, 0)))

    pcx, pcy, pw, ph = priors[:, 0], priors[:, 1], priors[:, 2], priors[:, 3]
    px1 = pcx - pw / 2.0
    py1 = pcy - ph / 2.0
    px2 = pcx + pw / 2.0
    py2 = pcy + ph / 2.0
    feats = jnp.stack([
        px1, py1, px2, py2,
        (px2 - px1) * (py2 - py1), pcx, pcy,
        1.0 / (_V0 * pw), 1.0 / (_V0 * ph),
        jnp.log(pw), jnp.log(ph),
    ], axis=0)
    pfeat = jnp.pad(feats, ((0, 16 - feats.shape[0]), (0, pp)))

    ll, lc, n = pl.pallas_call(
        _body,
        grid=(_B,),
        in_specs=[
            pl.BlockSpec((1, _O, 8), lambda i: (i, 0, 0)),
            pl.BlockSpec((1, 8, _O), lambda i: (i, 0, 0)),
            pl.BlockSpec((16, _P_PAD), lambda i: (0, 0)),
            pl.BlockSpec((1, 4, _P_PAD), lambda i: (i, 0, 0)),
            pl.BlockSpec((1, _C_PAD, _P_PAD), lambda i: (i, 0, 0)),
        ],
        out_specs=[
            pl.BlockSpec((1, 1), lambda i: (0, 0)),
            pl.BlockSpec((1, 1), lambda i: (0, 0)),
            pl.BlockSpec((1, 1), lambda i: (0, 0)),
        ],
        out_shape=[
            jax.ShapeDtypeStruct((1, 1), jnp.float32),
            jax.ShapeDtypeStruct((1, 1), jnp.float32),
            jax.ShapeDtypeStruct((1, 1), jnp.float32),
        ],
        scratch_shapes=[
            pltpu.VMEM((_B, _P_PAD), jnp.float32),
            pltpu.VMEM((_B, 128), jnp.float32),
        ],
    )(truths8, truthsT8, pfeat, locT, confT)

    N = n[0, 0]
    return ll[0, 0] / N, lc[0, 0] / N
